# baseline (speedup 1.0000x reference)
import functools

import jax
import jax.numpy as jnp
from jax import lax
from jax.experimental import pallas as pl
from jax.experimental.pallas import tpu as pltpu
from jax.experimental.pallas import tpu_sc as plsc

N = 10000
E = 320000
D = 128
NP = 10240
NC = 2
NS = 16
L = 16
CH = 64
NCH = E // CH
MCH = 313
CPC = NCH + 40
RPS = NP // NS
DEGR = NP // L
HG = 26


def _lin_body(x_ref, w_ref, b_ref, o_ref):
    o_ref[0] = (
        lax.dot_general(
            x_ref[...], w_ref[0], (((1,), (1,)), ((), ())),
            preferred_element_type=jnp.float32,
        )
        + b_ref[0]
    )


def _linear(x_pad, W_all, b_all):
    BM = 2048
    return pl.pallas_call(
        _lin_body,
        grid=(2, NP // BM),
        in_specs=[
            pl.BlockSpec((BM, D), lambda l, i: (i, 0)),
            pl.BlockSpec((1, D, D), lambda l, i: (l, 0, 0)),
            pl.BlockSpec((1, 1, D), lambda l, i: (l, 0, 0)),
        ],
        out_specs=pl.BlockSpec((1, BM, D), lambda l, i: (l, i, 0)),
        out_shape=jax.ShapeDtypeStruct((2, NP, D), jnp.float32),
    )(x_pad, W_all, b_all)


_mesh = plsc.VectorSubcoreMesh(
    core_axis_name="c", subcore_axis_name="s", num_cores=NC, num_subcores=NS
)


@functools.partial(
    pl.kernel,
    out_type=jax.ShapeDtypeStruct((2 * NP, D), jnp.float32),
    mesh=_mesh,
    compiler_params=pltpu.CompilerParams(
        needs_layout_passes=False, use_tc_tiling_on_sc=False
    ),
    scratch_types=[
        pltpu.VMEM_SHARED((DEGR, L), jnp.float32),
        pltpu.VMEM_SHARED((NP, D), jnp.float32),
        pltpu.VMEM((DEGR, L), jnp.float32),
        pltpu.VMEM((5, 128), jnp.int32),
        pltpu.VMEM((HG, CH), jnp.int32),
        pltpu.VMEM((6, CH), jnp.int32),
        pltpu.VMEM((2 * CH, D), jnp.float32),
        pltpu.VMEM((2 * CH, D), jnp.float32),
        pltpu.SemaphoreType.DMA,
        pltpu.SemaphoreType.DMA,
        pltpu.SemaphoreType.DMA,
    ],
)
def _sc_kernel(
    cols64, eidx, attr, hflat, zrows, zdeg, idrows, out,
    dcomp_sp, s_sp, dis_v, idr_v, colq_v, idx2_v, hrows_v, attr_v,
    sem_i, sem_g, sem_a,
):
    cid = lax.axis_index("c")
    sid = lax.axis_index("s")
    one16 = jnp.ones((L,), jnp.float32)

    pltpu.sync_copy(zdeg, dis_v)
    pltpu.sync_copy(idrows, idr_v)
    pltpu.sync_copy(zdeg.at[pl.ds(0, 40)], dcomp_sp.at[pl.ds(sid * 40, 40)])
    for k in range(5):
        pltpu.sync_copy(zrows, s_sp.at[pl.ds(sid * RPS + k * 128, 128)])
    plsc.subcore_barrier()

    def hist_rows(nrows):
        def hrow(a, carry):
            for bq in range(4):
                c16 = colq_v[a, pl.ds(bq * 16, 16)]
                hi = lax.shift_right_logical(c16, 4)
                lo = lax.bitwise_and(c16, 15)
                plsc.addupdate_scatter(dis_v, [hi, lo], one16)
            return carry

        lax.fori_loop(0, nrows, hrow, 0)

    def hgroup(g, carry):
        r = sid * 312 + g * HG
        pltpu.sync_copy(cols64.at[pl.ds(r, HG)], colq_v)
        hist_rows(HG)
        return carry

    lax.fori_loop(0, 12, hgroup, 0)

    @pl.when(sid == 15)
    def _():
        pltpu.sync_copy(cols64.at[pl.ds(4992, 8)], colq_v.at[pl.ds(0, 8)])
        hist_rows(8)

    for k in range(5):
        pltpu.sync_copy(
            dis_v.at[pl.ds(k * 128, 128)], dcomp_sp.at[idr_v.at[k]], add=True
        )
    plsc.subcore_barrier()

    pltpu.sync_copy(dcomp_sp, dis_v)

    def newton(i, carry):
        d = dis_v[i]
        ib = plsc.bitcast(d, jnp.int32)
        ib = 0x5F3759DF - lax.shift_right_logical(ib, 1)
        y = plsc.bitcast(ib, jnp.float32)
        y = y * (1.5 - 0.5 * d * y * y)
        y = y * (1.5 - 0.5 * d * y * y)
        y = y * (1.5 - 0.5 * d * y * y)
        dis_v[i] = jnp.where(d > 0.5, y, 0.0)
        return carry

    lax.fori_loop(0, DEGR, newton, 0)


    def issue_idx(m, s):
        rb = (cid * CPC + sid + 16 * m) * 3
        pltpu.async_copy(
            eidx.at[pl.ds(rb, 3)], idx2_v.at[pl.ds(3 * s, 3)], sem_i
        )

    def wait_idx(s):
        pltpu.make_async_copy(
            eidx.at[pl.ds(0, 3)], idx2_v.at[pl.ds(3 * s, 3)], sem_i
        ).wait()

    def issue_ga(m, s):
        c = jnp.minimum(sid + 16 * m, NCH - 1)
        base = pl.multiple_of(c * CH, CH)
        pltpu.async_copy(
            hflat.at[idx2_v.at[3 * s + 1]],
            hrows_v.at[pl.ds(CH * s, CH)],
            sem_g,
        )
        pltpu.async_copy(
            attr.at[pl.ds(base, CH)], attr_v.at[pl.ds(CH * s, CH)], sem_a
        )

    def wait_ga(s):
        pltpu.make_async_copy(
            attr.at[pl.ds(0, CH)], hrows_v.at[pl.ds(CH * s, CH)], sem_g
        ).wait()
        pltpu.make_async_copy(
            attr.at[pl.ds(0, CH)], attr_v.at[pl.ds(CH * s, CH)], sem_a
        ).wait()

    issue_idx(0, 0)
    wait_idx(0)
    issue_ga(0, 0)
    issue_idx(1, 1)

    def echunk(m, carry):
        s = lax.bitwise_and(m, 1)
        sn = 1 - s
        wait_idx(sn)
        wait_ga(s)
        issue_ga(m + 1, sn)

        def qloop(q, carry2):
            r16 = idx2_v[3 * s + 2, pl.ds(q * 16, 16)]
            hi = lax.shift_right_logical(r16, 4)
            lo = lax.bitwise_and(r16, 15)
            drv = plsc.load_gather(dis_v, [hi, lo])
            for j in range(16):
                b = jnp.broadcast_to(drv[j], (L,))
                e = CH * s + q * 16 + j
                for f in range(8):
                    sl = pl.ds(f * 16, 16)
                    attr_v[e, sl] = (attr_v[e, sl] + hrows_v[e, sl]) * b
            return carry2

        lax.fori_loop(0, 4, qloop, 0)
        pltpu.sync_copy(
            attr_v.at[pl.ds(CH * s, CH)], s_sp.at[idx2_v.at[3 * s]], add=True
        )
        issue_idx(m + 2, s)
        return carry

    lax.fori_loop(0, MCH, echunk, 0)
    wait_ga(MCH % 2)
    wait_idx((MCH + 1) % 2)

    plsc.subcore_barrier()

    for k in range(5):
        r0 = sid * RPS + k * 128
        pltpu.async_copy(s_sp.at[pl.ds(r0, 128)], hrows_v, sem_g)
        pltpu.async_copy(hflat.at[pl.ds(cid * NP + r0, 128)], attr_v, sem_a)
        pltpu.make_async_copy(attr.at[pl.ds(0, 128)], hrows_v, sem_g).wait()
        pltpu.make_async_copy(attr.at[pl.ds(0, 128)], attr_v, sem_a).wait()

        def wrow(j, carry):
            node = r0 + j
            hi = jnp.broadcast_to(lax.shift_right_logical(node, 4), (L,))
            lo = jnp.broadcast_to(lax.bitwise_and(node, 15), (L,))
            dn = plsc.load_gather(dis_v, [hi, lo])
            for f in range(8):
                sl = pl.ds(f * 16, 16)
                hrows_v[j, sl] = hrows_v[j, sl] * dn + attr_v[j, sl]
            return carry

        lax.fori_loop(0, 128, wrow, 0)
        pltpu.sync_copy(hrows_v, out.at[pl.ds(cid * NP + r0, 128)])


def kernel(x, edge_index, edge_attr, W_mean, b_mean, W_std, b_std):
    x_pad = jnp.pad(x, ((0, NP - N), (0, 0)))
    W_all = jnp.stack([W_mean, W_std])
    b_all = jnp.stack([b_mean, b_std])[:, None, :]
    h_all = _linear(x_pad, W_all, b_all)
    h_flat = h_all.reshape(2 * NP, D)
    rows64 = edge_index[0].reshape(NCH, CH)
    cols64 = edge_index[1].reshape(NCH, CH)
    rpad = jnp.zeros((CPC - NCH, CH), jnp.int32)
    cpad = jnp.full((CPC - NCH, CH), NP - 1, jnp.int32)
    rows64p = jnp.concatenate([rows64, rpad], axis=0)
    cols64p = jnp.concatenate([cols64, cpad], axis=0)
    eidx = jnp.concatenate(
        [
            jnp.stack([cols64p, rows64p, rows64p], axis=1),
            jnp.stack([cols64p, rows64p + NP, rows64p], axis=1),
        ],
        axis=0,
    ).reshape(2 * CPC * 3, CH)
    zrows = jnp.zeros((128, D), jnp.float32)
    zdeg = jnp.zeros((DEGR, L), jnp.float32)
    idrows = jnp.arange(5 * 128, dtype=jnp.int32).reshape(5, 128)
    out_flat = _sc_kernel(cols64, eidx, edge_attr, h_flat, zrows, zdeg, idrows)
    out = out_flat.reshape(2, NP, D)
    return out[0, :N], out[1, :N]


# speedup vs baseline: 1.0776x; 1.0776x over previous
import functools

import jax
import jax.numpy as jnp
from jax import lax
from jax.experimental import pallas as pl
from jax.experimental.pallas import tpu as pltpu
from jax.experimental.pallas import tpu_sc as plsc

N = 10000
E = 320000
D = 128
NP = 10240
NC = 2
NS = 16
L = 16
CH = 64
NCH = E // CH
MCH = 313
CPC = NCH + 40
RPS = NP // NS
DEGR = NP // L
HG = 26


def _lin_body(x_ref, w_ref, b_ref, o_ref):
    o_ref[0] = (
        lax.dot_general(
            x_ref[...], w_ref[0], (((1,), (1,)), ((), ())),
            preferred_element_type=jnp.float32,
        )
        + b_ref[0]
    )


def _linear(x_pad, W_all, b_all):
    BM = 2048
    return pl.pallas_call(
        _lin_body,
        grid=(2, NP // BM),
        in_specs=[
            pl.BlockSpec((BM, D), lambda l, i: (i, 0)),
            pl.BlockSpec((1, D, D), lambda l, i: (l, 0, 0)),
            pl.BlockSpec((1, 1, D), lambda l, i: (l, 0, 0)),
        ],
        out_specs=pl.BlockSpec((1, BM, D), lambda l, i: (l, i, 0)),
        out_shape=jax.ShapeDtypeStruct((2, NP, D), jnp.float32),
    )(x_pad, W_all, b_all)


_mesh = plsc.VectorSubcoreMesh(
    core_axis_name="c", subcore_axis_name="s", num_cores=NC, num_subcores=NS
)


@functools.partial(
    pl.kernel,
    out_type=jax.ShapeDtypeStruct((2 * NP, D), jnp.float32),
    mesh=_mesh,
    compiler_params=pltpu.CompilerParams(
        needs_layout_passes=False, use_tc_tiling_on_sc=False
    ),
    scratch_types=[
        pltpu.VMEM_SHARED((DEGR, L), jnp.float32),
        pltpu.VMEM_SHARED((NP, D), jnp.float32),
        pltpu.VMEM((DEGR, L), jnp.float32),
        pltpu.VMEM((5, 128), jnp.int32),
        pltpu.VMEM((HG, CH), jnp.int32),
        pltpu.VMEM((9, CH), jnp.int32),
        pltpu.VMEM((2 * CH, D), jnp.float32),
        pltpu.VMEM((2 * CH, D), jnp.float32),
        pltpu.SemaphoreType.DMA,
        pltpu.SemaphoreType.DMA,
        pltpu.SemaphoreType.DMA,
        pltpu.SemaphoreType.DMA,
    ],
)
def _sc_kernel(
    cols64, eidx, attr, hflat, zrows, zdeg, idrows, out,
    dcomp_sp, s_sp, dis_v, idr_v, colq_v, idx2_v, hrows_v, attr_v,
    sem_i, sem_g, sem_a, sem_s,
):
    cid = lax.axis_index("c")
    sid = lax.axis_index("s")
    one16 = jnp.ones((L,), jnp.float32)

    pltpu.sync_copy(zdeg, dis_v)
    pltpu.sync_copy(idrows, idr_v)
    pltpu.sync_copy(zdeg.at[pl.ds(0, 40)], dcomp_sp.at[pl.ds(sid * 40, 40)])
    for k in range(5):
        pltpu.sync_copy(zrows, s_sp.at[pl.ds(sid * RPS + k * 128, 128)])
    plsc.subcore_barrier()

    def hist_rows(nrows):
        def hrow(a, carry):
            for bq in range(4):
                c16 = colq_v[a, pl.ds(bq * 16, 16)]
                hi = lax.shift_right_logical(c16, 4)
                lo = lax.bitwise_and(c16, 15)
                plsc.addupdate_scatter(dis_v, [hi, lo], one16)
            return carry

        lax.fori_loop(0, nrows, hrow, 0)

    def hgroup(g, carry):
        r = sid * 312 + g * HG
        pltpu.sync_copy(cols64.at[pl.ds(r, HG)], colq_v)
        hist_rows(HG)
        return carry

    lax.fori_loop(0, 12, hgroup, 0)

    @pl.when(sid == 15)
    def _():
        pltpu.sync_copy(cols64.at[pl.ds(4992, 8)], colq_v.at[pl.ds(0, 8)])
        hist_rows(8)

    for k in range(5):
        pltpu.sync_copy(
            dis_v.at[pl.ds(k * 128, 128)], dcomp_sp.at[idr_v.at[k]], add=True
        )
    plsc.subcore_barrier()

    pltpu.sync_copy(dcomp_sp, dis_v)

    def newton(i, carry):
        d = dis_v[i]
        ib = plsc.bitcast(d, jnp.int32)
        ib = 0x5F3759DF - lax.shift_right_logical(ib, 1)
        y = plsc.bitcast(ib, jnp.float32)
        y = y * (1.5 - 0.5 * d * y * y)
        y = y * (1.5 - 0.5 * d * y * y)
        y = y * (1.5 - 0.5 * d * y * y)
        dis_v[i] = jnp.where(d > 0.5, y, 0.0)
        return carry

    lax.fori_loop(0, DEGR, newton, 0)


    def issue_idx(m, si):
        rb = (cid * CPC + sid + 16 * m) * 3
        pltpu.async_copy(
            eidx.at[pl.ds(rb, 3)], idx2_v.at[pl.ds(3 * si, 3)], sem_i
        )

    def wait_idx(si):
        pltpu.make_async_copy(
            eidx.at[pl.ds(0, 3)], idx2_v.at[pl.ds(3 * si, 3)], sem_i
        ).wait()

    def issue_ga(m, si, sd):
        c = jnp.minimum(sid + 16 * m, NCH - 1)
        base = pl.multiple_of(c * CH, CH)
        pltpu.async_copy(
            hflat.at[idx2_v.at[3 * si + 1]],
            hrows_v.at[pl.ds(CH * sd, CH)],
            sem_g,
        )
        pltpu.async_copy(
            attr.at[pl.ds(base, CH)], attr_v.at[pl.ds(CH * sd, CH)], sem_a
        )

    def wait_ga(sd):
        pltpu.make_async_copy(
            attr.at[pl.ds(0, CH)], hrows_v.at[pl.ds(CH * sd, CH)], sem_g
        ).wait()
        pltpu.make_async_copy(
            attr.at[pl.ds(0, CH)], attr_v.at[pl.ds(CH * sd, CH)], sem_a
        ).wait()

    def issue_s(si, sd):
        pltpu.async_copy(
            attr_v.at[pl.ds(CH * sd, CH)],
            s_sp.at[idx2_v.at[3 * si]],
            sem_s,
            add=True,
        )

    def wait_s(si, sd):
        pltpu.make_async_copy(
            attr_v.at[pl.ds(CH * sd, CH)], s_sp.at[idx2_v.at[3 * si]], sem_s
        ).wait()

    issue_idx(0, 0)
    wait_idx(0)
    issue_ga(0, 0, 0)
    issue_idx(1, 1)

    def echunk(m, carry):
        sd = lax.bitwise_and(m, 1)
        sdn = 1 - sd
        si = lax.rem(m, 3)
        sin = lax.rem(m + 1, 3)
        sip = lax.rem(m + 2, 3)
        wait_idx(sin)
        wait_ga(sd)

        @pl.when(m > 0)
        def _():
            wait_s(lax.rem(m + 2, 3), sdn)

        issue_ga(m + 1, sin, sdn)

        def qloop(q, carry2):
            r16 = idx2_v[3 * si + 2, pl.ds(q * 16, 16)]
            hi = lax.shift_right_logical(r16, 4)
            lo = lax.bitwise_and(r16, 15)
            drv = plsc.load_gather(dis_v, [hi, lo])
            for j in range(16):
                b = jnp.broadcast_to(drv[j], (L,))
                e = CH * sd + q * 16 + j
                for f in range(8):
                    sl = pl.ds(f * 16, 16)
                    attr_v[e, sl] = (attr_v[e, sl] + hrows_v[e, sl]) * b
            return carry2

        lax.fori_loop(0, 4, qloop, 0)
        issue_s(si, sd)
        issue_idx(m + 2, sip)
        return carry

    lax.fori_loop(0, MCH, echunk, 0)
    wait_s((MCH - 1) % 3, (MCH - 1) % 2)
    wait_ga(MCH % 2)
    wait_idx((MCH + 1) % 3)

    plsc.subcore_barrier()

    for k in range(5):
        r0 = sid * RPS + k * 128
        pltpu.async_copy(s_sp.at[pl.ds(r0, 128)], hrows_v, sem_g)
        pltpu.async_copy(hflat.at[pl.ds(cid * NP + r0, 128)], attr_v, sem_a)
        pltpu.make_async_copy(attr.at[pl.ds(0, 128)], hrows_v, sem_g).wait()
        pltpu.make_async_copy(attr.at[pl.ds(0, 128)], attr_v, sem_a).wait()

        def wrow(j, carry):
            node = r0 + j
            hi = jnp.broadcast_to(lax.shift_right_logical(node, 4), (L,))
            lo = jnp.broadcast_to(lax.bitwise_and(node, 15), (L,))
            dn = plsc.load_gather(dis_v, [hi, lo])
            for f in range(8):
                sl = pl.ds(f * 16, 16)
                hrows_v[j, sl] = hrows_v[j, sl] * dn + attr_v[j, sl]
            return carry

        lax.fori_loop(0, 128, wrow, 0)
        pltpu.sync_copy(hrows_v, out.at[pl.ds(cid * NP + r0, 128)])


def kernel(x, edge_index, edge_attr, W_mean, b_mean, W_std, b_std):
    x_pad = jnp.pad(x, ((0, NP - N), (0, 0)))
    W_all = jnp.stack([W_mean, W_std])
    b_all = jnp.stack([b_mean, b_std])[:, None, :]
    h_all = _linear(x_pad, W_all, b_all)
    h_flat = h_all.reshape(2 * NP, D)
    rows64 = edge_index[0].reshape(NCH, CH)
    cols64 = edge_index[1].reshape(NCH, CH)
    rpad = jnp.zeros((CPC - NCH, CH), jnp.int32)
    cpad = jnp.full((CPC - NCH, CH), NP - 1, jnp.int32)
    rows64p = jnp.concatenate([rows64, rpad], axis=0)
    cols64p = jnp.concatenate([cols64, cpad], axis=0)
    eidx = jnp.concatenate(
        [
            jnp.stack([cols64p, rows64p, rows64p], axis=1),
            jnp.stack([cols64p, rows64p + NP, rows64p], axis=1),
        ],
        axis=0,
    ).reshape(2 * CPC * 3, CH)
    zrows = jnp.zeros((128, D), jnp.float32)
    zdeg = jnp.zeros((DEGR, L), jnp.float32)
    idrows = jnp.arange(5 * 128, dtype=jnp.int32).reshape(5, 128)
    out_flat = _sc_kernel(cols64, eidx, edge_attr, h_flat, zrows, zdeg, idrows)
    out = out_flat.reshape(2, NP, D)
    return out[0, :N], out[1, :N]


# speedup vs baseline: 1.5992x; 1.4840x over previous
import functools

import jax
import jax.numpy as jnp
from jax import lax
from jax.experimental import pallas as pl
from jax.experimental.pallas import tpu as pltpu
from jax.experimental.pallas import tpu_sc as plsc

N = 10000
E = 320000
D = 128
NP = 10240
NC = 2
NS = 16
L = 16
CH = 128
NCH = E // CH
MCH = 157
CPC = NCH + 44
NCC = E // 64
RPS = NP // NS
DEGR = NP // L
HG = 26


def _lin_body(x_ref, w_ref, b_ref, o_ref):
    o_ref[0] = (
        lax.dot_general(
            x_ref[...], w_ref[0], (((1,), (1,)), ((), ())),
            preferred_element_type=jnp.float32,
        )
        + b_ref[0]
    )


def _linear(x_pad, W_all, b_all):
    BM = 2048
    return pl.pallas_call(
        _lin_body,
        grid=(2, NP // BM),
        in_specs=[
            pl.BlockSpec((BM, D), lambda l, i: (i, 0)),
            pl.BlockSpec((1, D, D), lambda l, i: (l, 0, 0)),
            pl.BlockSpec((1, 1, D), lambda l, i: (l, 0, 0)),
        ],
        out_specs=pl.BlockSpec((1, BM, D), lambda l, i: (l, i, 0)),
        out_shape=jax.ShapeDtypeStruct((2, NP, D), jnp.float32),
    )(x_pad, W_all, b_all)


_mesh = plsc.VectorSubcoreMesh(
    core_axis_name="c", subcore_axis_name="s", num_cores=NC, num_subcores=NS
)


@functools.partial(
    pl.kernel,
    out_type=jax.ShapeDtypeStruct((2 * NP, D), jnp.float32),
    mesh=_mesh,
    compiler_params=pltpu.CompilerParams(
        needs_layout_passes=False, use_tc_tiling_on_sc=False
    ),
    scratch_types=[
        pltpu.VMEM_SHARED((DEGR, L), jnp.float32),
        pltpu.VMEM_SHARED((NP, D), jnp.float32),
        pltpu.VMEM((DEGR, L), jnp.float32),
        pltpu.VMEM((5, 128), jnp.int32),
        pltpu.VMEM((HG, 64), jnp.int32),
        pltpu.VMEM((10, CH), jnp.int32),
        pltpu.VMEM((CH, D), jnp.float32),
        pltpu.VMEM((CH, D), jnp.float32),
        pltpu.SemaphoreType.DMA,
        pltpu.SemaphoreType.DMA,
        pltpu.SemaphoreType.DMA,
        pltpu.SemaphoreType.DMA,
    ],
)
def _sc_kernel(
    cols64, eidx, attr, hflat, zrows, zdeg, idrows, out,
    dcomp_sp, s_sp, dis_v, idr_v, colq_v, idx2_v, hrows_v, attr_v,
    sem_i, sem_g, sem_a, sem_s,
):
    cid = lax.axis_index("c")
    sid = lax.axis_index("s")
    one16 = jnp.ones((L,), jnp.float32)

    pltpu.sync_copy(zdeg, dis_v)
    pltpu.sync_copy(idrows, idr_v)
    pltpu.sync_copy(zdeg.at[pl.ds(0, 40)], dcomp_sp.at[pl.ds(sid * 40, 40)])
    for k in range(5):
        pltpu.sync_copy(zrows, s_sp.at[pl.ds(sid * RPS + k * 128, 128)])
    plsc.subcore_barrier()

    def hist_rows(nrows):
        def hrow(a, carry):
            for bq in range(4):
                c16 = colq_v[a, pl.ds(bq * 16, 16)]
                hi = lax.shift_right_logical(c16, 4)
                lo = lax.bitwise_and(c16, 15)
                plsc.addupdate_scatter(dis_v, [hi, lo], one16)
            return carry

        lax.fori_loop(0, nrows, hrow, 0)

    def hgroup(g, carry):
        r = sid * 312 + g * HG
        pltpu.sync_copy(cols64.at[pl.ds(r, HG)], colq_v)
        hist_rows(HG)
        return carry

    lax.fori_loop(0, 12, hgroup, 0)

    @pl.when(sid == 15)
    def _():
        pltpu.sync_copy(cols64.at[pl.ds(4992, 8)], colq_v.at[pl.ds(0, 8)])
        hist_rows(8)

    for k in range(5):
        pltpu.sync_copy(
            dis_v.at[pl.ds(k * 128, 128)], dcomp_sp.at[idr_v.at[k]], add=True
        )
    plsc.subcore_barrier()

    pltpu.sync_copy(dcomp_sp, dis_v)

    def newton(i, carry):
        d = dis_v[i]
        ib = plsc.bitcast(d, jnp.int32)
        ib = 0x5F3759DF - lax.shift_right_logical(ib, 1)
        y = plsc.bitcast(ib, jnp.float32)
        y = y * (1.5 - 0.5 * d * y * y)
        y = y * (1.5 - 0.5 * d * y * y)
        y = y * (1.5 - 0.5 * d * y * y)
        dis_v[i] = jnp.where(d > 0.5, y, 0.0)
        return carry

    lax.fori_loop(0, DEGR, newton, 0)


    def issue_idx(m, si):
        rb = (cid * CPC + sid + 16 * m) * 3
        pltpu.async_copy(
            eidx.at[pl.ds(rb, 3)], idx2_v.at[pl.ds(3 * si, 3)], sem_i
        )

    def wait_idx(si):
        pltpu.make_async_copy(
            eidx.at[pl.ds(0, 3)], idx2_v.at[pl.ds(3 * si, 3)], sem_i
        ).wait()

    def issue_ga(m, si):
        c = jnp.minimum(sid + 16 * m, NCH - 1)
        base = pl.multiple_of(c * CH, CH)
        pltpu.async_copy(
            hflat.at[idx2_v.at[3 * si + 1]], hrows_v, sem_g
        )
        pltpu.async_copy(attr.at[pl.ds(base, CH)], attr_v, sem_a)

    def wait_ga():
        pltpu.make_async_copy(attr.at[pl.ds(0, CH)], hrows_v, sem_g).wait()
        pltpu.make_async_copy(attr.at[pl.ds(0, CH)], attr_v, sem_a).wait()

    def issue_s(si):
        pltpu.async_copy(
            attr_v, s_sp.at[idx2_v.at[3 * si]], sem_s, add=True
        )

    def wait_s(si):
        pltpu.make_async_copy(
            attr_v, s_sp.at[idx2_v.at[3 * si]], sem_s
        ).wait()

    pltpu.async_copy(
        eidx.at[pl.ds((cid * CPC + NCH + sid) * 3, 3)],
        idx2_v.at[pl.ds(6, 3)],
        sem_i,
    )
    wait_idx(2)
    issue_s(2)
    issue_idx(0, 0)

    def echunk(m, carry):
        si = lax.rem(m, 3)
        sin = lax.rem(m + 1, 3)
        wait_idx(si)
        issue_idx(m + 1, sin)
        wait_s(lax.rem(m + 2, 3))
        issue_ga(m, si)
        wait_ga()

        def qloop(q, carry2):
            r16 = idx2_v[3 * si + 2, pl.ds(q * 8, 16)]
            hi = lax.bitwise_and(lax.shift_right_logical(r16, 4), 1023)
            lo = lax.bitwise_and(r16, 15)
            drv = plsc.load_gather(dis_v, [hi, lo])
            for j in range(8):
                b = jnp.broadcast_to(drv[j], (L,))
                e = q * 8 + j

                def fbody(f, c3, _e=e, _b=b):
                    for ff in range(2):
                        sl = pl.ds((f * 2 + ff) * 16, 16)
                        attr_v[_e, sl] = (
                            attr_v[_e, sl] + hrows_v[_e, sl]
                        ) * _b
                    return c3

                lax.fori_loop(0, 4, fbody, 0)
            return carry2

        lax.fori_loop(0, 16, qloop, 0)
        issue_s(si)
        return carry

    lax.fori_loop(0, MCH, echunk, 0)
    wait_s((MCH - 1) % 3)
    wait_idx(MCH % 3)

    plsc.subcore_barrier()

    for k in range(5):
        r0 = sid * RPS + k * 128
        pltpu.async_copy(s_sp.at[pl.ds(r0, 128)], hrows_v, sem_g)
        pltpu.async_copy(hflat.at[pl.ds(cid * NP + r0, 128)], attr_v, sem_a)
        pltpu.make_async_copy(attr.at[pl.ds(0, 128)], hrows_v, sem_g).wait()
        pltpu.make_async_copy(attr.at[pl.ds(0, 128)], attr_v, sem_a).wait()

        def wrow(j, carry):
            node = r0 + j
            hi = jnp.broadcast_to(lax.shift_right_logical(node, 4), (L,))
            lo = jnp.broadcast_to(lax.bitwise_and(node, 15), (L,))
            dn = plsc.load_gather(dis_v, [hi, lo])
            for f in range(8):
                sl = pl.ds(f * 16, 16)
                hrows_v[j, sl] = hrows_v[j, sl] * dn + attr_v[j, sl]
            return carry

        lax.fori_loop(0, 128, wrow, 0)
        pltpu.sync_copy(hrows_v, out.at[pl.ds(cid * NP + r0, 128)])


def kernel(x, edge_index, edge_attr, W_mean, b_mean, W_std, b_std):
    x_pad = jnp.pad(x, ((0, NP - N), (0, 0)))
    W_all = jnp.stack([W_mean, W_std])
    b_all = jnp.stack([b_mean, b_std])[:, None, :]
    h_all = _linear(x_pad, W_all, b_all)
    h_flat = h_all.reshape(2 * NP, D)
    rowsc = edge_index[0].reshape(NCH, CH)
    colsc = edge_index[1].reshape(NCH, CH)
    cols64 = edge_index[1].reshape(NCC, 64)
    rpad = jnp.zeros((CPC - NCH, CH), jnp.int32)
    cpad = jnp.full((CPC - NCH, CH), NP - 1, jnp.int32)
    rowsp = jnp.concatenate([rowsc, rpad], axis=0)
    colsp = jnp.concatenate([colsc, cpad], axis=0)
    eidx = jnp.concatenate(
        [
            jnp.stack([colsp, rowsp, rowsp], axis=1),
            jnp.stack([colsp, rowsp + NP, rowsp], axis=1),
        ],
        axis=0,
    ).reshape(2 * CPC * 3, CH)
    zrows = jnp.zeros((128, D), jnp.float32)
    zdeg = jnp.zeros((DEGR, L), jnp.float32)
    idrows = jnp.arange(5 * 128, dtype=jnp.int32).reshape(5, 128)
    out_flat = _sc_kernel(cols64, eidx, edge_attr, h_flat, zrows, zdeg, idrows)
    out = out_flat.reshape(2, NP, D)
    return out[0, :N], out[1, :N]
